# Initial kernel scaffold; baseline (speedup 1.0000x reference)
import jax
import jax.numpy as jnp
from jax.experimental import pallas as pl


def kernel(x, table):
    raise NotImplementedError("write your pallas kernel here")


# speedup vs baseline: 1.7354x; 1.7354x over previous
import functools

import jax
import jax.numpy as jnp
from jax import lax
from jax.experimental import pallas as pl
from jax.experimental.pallas import tpu as pltpu
from jax.experimental.pallas import tpu_sc as plsc

VOCAB = 8192
D = 8192
B = 4096

NC = 2
NS = 16
NW = NC * NS
ROWS_PER_W = B // NW
CH = 8
NCHUNK = ROWS_PER_W // CH


def _sc_gather(table, idx):
    mesh = plsc.VectorSubcoreMesh(core_axis_name="c", subcore_axis_name="s")

    @functools.partial(
        pl.kernel,
        mesh=mesh,
        out_type=jax.ShapeDtypeStruct((B, D), jnp.float32),
        scratch_types=[
            pltpu.VMEM((NCHUNK, CH), jnp.int32),
            pltpu.VMEM((CH, D), jnp.float32),
            pltpu.SemaphoreType.DMA,
        ],
    )
    def k(table_hbm, idx_hbm, out_hbm, idx_v, buf, gsem):
        wid = lax.axis_index("s") * NC + lax.axis_index("c")
        pltpu.sync_copy(idx_hbm.at[wid], idx_v)
        base = wid * ROWS_PER_W

        def body(i, carry):
            pltpu.async_copy(table_hbm.at[idx_v.at[i]], buf, gsem).wait()
            pltpu.sync_copy(buf, out_hbm.at[pl.ds(base + i * CH, CH)])
            return carry

        lax.fori_loop(0, NCHUNK, body, 0)

    return k(table, idx)


def kernel(x, table):
    idx = x.reshape(-1).astype(jnp.int32).reshape(NW, NCHUNK, CH)
    return _sc_gather(table, idx)


# speedup vs baseline: 1.8147x; 1.0457x over previous
import functools

import jax
import jax.numpy as jnp
from jax import lax
from jax.experimental import pallas as pl
from jax.experimental.pallas import tpu as pltpu
from jax.experimental.pallas import tpu_sc as plsc

VOCAB = 8192
D = 8192
B = 4096

NC = 2
NS = 16
NW = NC * NS
ROWS_PER_W = B // NW
CH = 4
NCHUNK = ROWS_PER_W // CH
NGROUP = NCHUNK // 2


def _sc_gather(table, idx):
    mesh = plsc.VectorSubcoreMesh(core_axis_name="c", subcore_axis_name="s")

    @functools.partial(
        pl.kernel,
        mesh=mesh,
        out_type=jax.ShapeDtypeStruct((B, D), jnp.float32),
        scratch_types=[
            pltpu.VMEM((NCHUNK, CH), jnp.int32),
            pltpu.VMEM((CH, D), jnp.float32),
            pltpu.VMEM((CH, D), jnp.float32),
            pltpu.SemaphoreType.DMA,
            pltpu.SemaphoreType.DMA,
            pltpu.SemaphoreType.DMA,
            pltpu.SemaphoreType.DMA,
        ],
    )
    def k(table_hbm, idx_hbm, out_hbm, idx_v, buf0, buf1, g0, g1, w0, w1):
        wid = lax.axis_index("s") * NC + lax.axis_index("c")
        pltpu.sync_copy(idx_hbm.at[wid], idx_v)
        base = wid * ROWS_PER_W

        def gather(i, buf, sem):
            pltpu.async_copy(table_hbm.at[idx_v.at[i]], buf, sem)

        def write(i, buf, sem):
            pltpu.async_copy(buf, out_hbm.at[pl.ds(base + i * CH, CH)], sem)

        gather(0, buf0, g0)

        def body(g, carry):
            i0 = 2 * g
            pltpu.make_async_copy(table_hbm.at[idx_v.at[i0]], buf0, g0).wait()

            @pl.when(g > 0)
            def _():
                pltpu.make_async_copy(
                    buf1, out_hbm.at[pl.ds(base + (i0 - 1) * CH, CH)], w1
                ).wait()

            gather(i0 + 1, buf1, g1)
            write(i0, buf0, w0)
            pltpu.make_async_copy(table_hbm.at[idx_v.at[i0 + 1]], buf1, g1).wait()
            pltpu.make_async_copy(
                buf0, out_hbm.at[pl.ds(base + i0 * CH, CH)], w0
            ).wait()

            @pl.when(g + 1 < NGROUP)
            def _():
                gather(i0 + 2, buf0, g0)

            write(i0 + 1, buf1, w1)
            return carry

        lax.fori_loop(0, NGROUP, body, 0)
        pltpu.make_async_copy(
            buf1, out_hbm.at[pl.ds(base + (NCHUNK - 1) * CH, CH)], w1
        ).wait()

    return k(table, idx)


def kernel(x, table):
    idx = x.reshape(-1).astype(jnp.int32).reshape(NW, NCHUNK, CH)
    return _sc_gather(table, idx)


# speedup vs baseline: 1.8744x; 1.0329x over previous
import functools

import jax
import jax.numpy as jnp
from jax import lax
from jax.experimental import pallas as pl
from jax.experimental.pallas import tpu as pltpu
from jax.experimental.pallas import tpu_sc as plsc

VOCAB = 8192
D = 8192
B = 4096

NC = 2
NS = 16
NW = NC * NS
ROWS_PER_W = B // NW
CH = 2
NCHUNK = ROWS_PER_W // CH
NBUF = 4
NGROUP = NCHUNK // NBUF


def _sc_gather(table, idx):
    mesh = plsc.VectorSubcoreMesh(core_axis_name="c", subcore_axis_name="s")

    @functools.partial(
        pl.kernel,
        mesh=mesh,
        out_type=jax.ShapeDtypeStruct((B, D), jnp.float32),
        scratch_types=[
            pltpu.VMEM((NCHUNK, CH), jnp.int32),
        ]
        + [pltpu.VMEM((CH, D), jnp.float32)] * NBUF
        + [pltpu.SemaphoreType.DMA] * (2 * NBUF),
    )
    def k(table_hbm, idx_hbm, out_hbm, idx_v, *bufs_sems):
        bufs = bufs_sems[:NBUF]
        gsem = bufs_sems[NBUF : 2 * NBUF]
        wsem = bufs_sems[2 * NBUF :]
        wid = lax.axis_index("s") * NC + lax.axis_index("c")
        pltpu.sync_copy(idx_hbm.at[wid], idx_v)
        base = wid * ROWS_PER_W

        def gather(i, b):
            pltpu.async_copy(table_hbm.at[idx_v.at[i]], bufs[b], gsem[b])

        def wait_gather(i, b):
            pltpu.make_async_copy(table_hbm.at[idx_v.at[i]], bufs[b], gsem[b]).wait()

        def write(i, b):
            pltpu.async_copy(bufs[b], out_hbm.at[pl.ds(base + i * CH, CH)], wsem[b])

        def wait_write(i, b):
            pltpu.make_async_copy(
                bufs[b], out_hbm.at[pl.ds(base + i * CH, CH)], wsem[b]
            ).wait()

        gather(0, 0)
        gather(1, 1)

        def body(g, carry):
            i0 = NBUF * g
            for b in range(NBUF):
                i = i0 + b
                b2 = (b + 2) % NBUF
                wait_gather(i, b)
                write(i, b)
                @pl.when(i + 2 < NCHUNK)
                def _(i=i, b2=b2):
                    @pl.when(i >= 2)
                    def _():
                        wait_write(i - 2, b2)

                    gather(i + 2, b2)

            return carry

        lax.fori_loop(0, NGROUP, body, 0)
        for i in range(NCHUNK - 4, NCHUNK):
            wait_write(i, i % NBUF)

    return k(table, idx)


def kernel(x, table):
    idx = x.reshape(-1).astype(jnp.int32).reshape(NW, NCHUNK, CH)
    return _sc_gather(table, idx)


# speedup vs baseline: 1.9074x; 1.0176x over previous
import functools

import jax
import jax.numpy as jnp
from jax import lax
from jax.experimental import pallas as pl
from jax.experimental.pallas import tpu as pltpu
from jax.experimental.pallas import tpu_sc as plsc

VOCAB = 8192
D = 8192
B = 4096

NC = 2
NS = 16
NW = NC * NS
ROWS_PER_W = B // NW
CH = 1
NCHUNK = ROWS_PER_W // CH
NBUF = 8
LA = 4
NGROUP = NCHUNK // NBUF


def _sc_gather(table, idx):
    mesh = plsc.VectorSubcoreMesh(core_axis_name="c", subcore_axis_name="s")

    @functools.partial(
        pl.kernel,
        mesh=mesh,
        out_type=jax.ShapeDtypeStruct((B, D), jnp.float32),
        scratch_types=[
            pltpu.VMEM((NCHUNK, CH), jnp.int32),
        ]
        + [pltpu.VMEM((CH, D), jnp.float32)] * NBUF
        + [pltpu.SemaphoreType.DMA] * (2 * NBUF),
    )
    def k(table_hbm, idx_hbm, out_hbm, idx_v, *bufs_sems):
        bufs = bufs_sems[:NBUF]
        gsem = bufs_sems[NBUF : 2 * NBUF]
        wsem = bufs_sems[2 * NBUF :]
        wid = lax.axis_index("s") * NC + lax.axis_index("c")
        pltpu.sync_copy(idx_hbm.at[wid], idx_v)
        base = wid * ROWS_PER_W

        def gather(i, b):
            pltpu.async_copy(table_hbm.at[idx_v.at[i]], bufs[b], gsem[b])

        def wait_gather(i, b):
            pltpu.make_async_copy(table_hbm.at[idx_v.at[i]], bufs[b], gsem[b]).wait()

        def write(i, b):
            pltpu.async_copy(bufs[b], out_hbm.at[pl.ds(base + i * CH, CH)], wsem[b])

        def wait_write(i, b):
            pltpu.make_async_copy(
                bufs[b], out_hbm.at[pl.ds(base + i * CH, CH)], wsem[b]
            ).wait()

        for i in range(LA):
            gather(i, i)

        def body(g, carry):
            i0 = NBUF * g
            for b in range(NBUF):
                i = i0 + b
                b2 = (b + LA) % NBUF
                wait_gather(i, b)
                write(i, b)
                @pl.when(i + LA < NCHUNK)
                def _(i=i, b2=b2):
                    @pl.when(i + LA >= NBUF)
                    def _():
                        wait_write(i + LA - NBUF, b2)

                    gather(i + LA, b2)

            return carry

        lax.fori_loop(0, NGROUP, body, 0)
        for i in range(NCHUNK - NBUF, NCHUNK):
            wait_write(i, i % NBUF)

    return k(table, idx)


def kernel(x, table):
    idx = x.reshape(-1).astype(jnp.int32).reshape(NW, NCHUNK, CH)
    return _sc_gather(table, idx)
